# baseline (speedup 1.0000x reference)
import jax
import jax.numpy as jnp
from jax.experimental import pallas as pl
from jax.experimental.pallas import tpu as pltpu

_K = 144
_M = 392


def _bigru_kernel(xs_ref, ms_ref, w_ref, wx_ref, bout_ref,
                  out_ref, hx_ref, pf_ref, pb_ref):
    S = xs_ref.shape[0]
    H = wx_ref.shape[0]
    RT = xs_ref.shape[1]

    w = w_ref[:, :]
    wxh_f = wx_ref[:, 0:1]
    bxh_f = wx_ref[:, 1:2]
    wxh_b = wx_ref[:, 2:3]
    bxh_b = wx_ref[:, 3:4]
    wof = wx_ref[:, 4:5]
    wob = wx_ref[:, 5:6]
    half = jnp.bfloat16(0.5)

    hx_ref[:, :] = jnp.zeros_like(hx_ref)
    ones_pad = jnp.concatenate(
        [jnp.ones((1, RT), jnp.float32), jnp.zeros((1, RT), jnp.float32)],
        axis=0)
    hx_ref[pl.ds(2 * H + 2, 2), :] = ones_pad.astype(jnp.bfloat16)

    def step(t, carry):
        tb = S - 1 - t
        xf = xs_ref[pl.ds(t, 1), :].astype(jnp.bfloat16)
        xb = xs_ref[pl.ds(tb, 1), :].astype(jnp.bfloat16)
        hx_ref[pl.ds(2 * H, 2), :] = jnp.concatenate([xf, xb], axis=0)
        gf32 = jnp.dot(w, hx_ref[:, :],
                       preferred_element_type=jnp.float32)
        g = gf32[0:6 * H, :].astype(jnp.bfloat16)

        tpf = jnp.maximum(t - 1, 0)
        tpb = jnp.minimum(S - t, S - 1)
        pf_ref[pl.ds(tpf, 1), :] = gf32[6 * H:6 * H + 1, :]
        pb_ref[pl.ds(tpb, 1), :] = gf32[6 * H + 1:6 * H + 2, :]

        th_f = jnp.tanh(g[0:2 * H, :])
        hhp_f = g[2 * H:3 * H, :]
        xh_f = wxh_f * xf + bxh_f
        cf = jnp.tanh(xh_f + hhp_f + hhp_f * th_f[H:2 * H, :])
        hf = hx_ref[pl.ds(0, H), :]
        hfn = half * (hf + cf + th_f[0:H, :] * (cf - hf))

        th_b = jnp.tanh(g[3 * H:5 * H, :])
        hhp_b = g[5 * H:6 * H, :]
        xh_b = wxh_b * xb + bxh_b
        cb = jnp.tanh(xh_b + hhp_b + hhp_b * th_b[H:2 * H, :])
        hb = hx_ref[pl.ds(H, H), :]
        hbn = half * (hb + cb + th_b[0:H, :] * (cb - hb))

        hx_ref[pl.ds(0, H), :] = hfn
        hx_ref[pl.ds(H, H), :] = hbn
        return carry

    jax.lax.fori_loop(0, S, step, 0)

    hfin = hx_ref[pl.ds(0, 2 * H), :]
    pf_ref[pl.ds(S - 1, 1), :] = jnp.sum(
        hfin[0:H, :] * wof, axis=0, keepdims=True).astype(jnp.float32)
    pb_ref[pl.ds(0, 1), :] = jnp.sum(
        hfin[H:2 * H, :] * wob, axis=0, keepdims=True).astype(jnp.float32)

    xs = xs_ref[:, :]
    m = ms_ref[:, :]
    imp = pf_ref[:, :] + pb_ref[:, :] + bout_ref[0, 0]
    out_ref[:, :] = m * xs + (1.0 - m) * imp


def _pack_weights(Wf, Uf, bf, Wb, Ub, bb, Wout, H):
    w = jnp.zeros((_M, _K), jnp.float32)
    UfT, UbT = Uf.T, Ub.T
    w = w.at[0:3 * H, 0:H].set(0.5 * UfT)
    w = w.at[3 * H:6 * H, H:2 * H].set(0.5 * UbT)
    w = w.at[0:2 * H, 2 * H].set(0.5 * Wf[0, 0:2 * H])
    w = w.at[3 * H:5 * H, 2 * H + 1].set(0.5 * Wb[0, 0:2 * H])
    w = w.at[0:2 * H, 2 * H + 2].set(0.5 * bf[0:2 * H])
    w = w.at[3 * H:5 * H, 2 * H + 2].set(0.5 * bb[0:2 * H])
    w = w.at[6 * H, 0:H].set(Wout[0:H, 0])
    w = w.at[6 * H + 1, H:2 * H].set(Wout[H:2 * H, 0])
    return w.astype(jnp.bfloat16)


def kernel(x, mask, Wf, Uf, bf, Wb, Ub, bb, Wout, bout):
    B, S, N, C = x.shape
    H = Uf.shape[0]
    R = B * N
    RT = 2048
    G = R // RT

    xs = x.transpose(1, 0, 2, 3).reshape(S, R)
    ms = mask.astype(jnp.float32).transpose(1, 0, 2, 3).reshape(S, R)

    w = _pack_weights(Wf, Uf, bf, Wb, Ub, bb, Wout, H)
    wx = jnp.stack([Wf[0, 2 * H:3 * H], bf[2 * H:3 * H],
                    Wb[0, 2 * H:3 * H], bb[2 * H:3 * H],
                    Wout[0:H, 0], Wout[H:2 * H, 0]],
                   axis=1).astype(jnp.bfloat16)
    bout2 = bout.reshape(1, 1)

    full = lambda shape: pl.BlockSpec(shape, lambda i: (0, 0))
    tile = pl.BlockSpec((S, RT), lambda i: (0, i))

    out = pl.pallas_call(
        _bigru_kernel,
        grid=(G,),
        in_specs=[
            tile,
            tile,
            full((_M, _K)),
            full((H, 6)),
            full((1, 1)),
        ],
        out_specs=tile,
        out_shape=jax.ShapeDtypeStruct((S, R), jnp.float32),
        scratch_shapes=[
            pltpu.VMEM((_K, RT), jnp.bfloat16),
            pltpu.VMEM((S, RT), jnp.float32),
            pltpu.VMEM((S, RT), jnp.float32),
        ],
        compiler_params=pltpu.CompilerParams(
            dimension_semantics=("arbitrary",),
        ),
    )(xs, ms, w, wx, bout2)

    return out.reshape(S, B, N, C).transpose(1, 0, 2, 3)


# speedup vs baseline: 1.1264x; 1.1264x over previous
import jax
import jax.numpy as jnp
from jax.experimental import pallas as pl
from jax.experimental.pallas import tpu as pltpu

_K = 144
_M = 392


def _bigru_kernel(xs_ref, ms_ref, w_ref, wx_ref, bout_ref,
                  out_ref, hx_ref, pf_ref, pb_ref):
    S = xs_ref.shape[0]
    H = wx_ref.shape[0]
    RT = xs_ref.shape[1]

    w = w_ref[:, :]
    wxh_f = wx_ref[:, 0:1]
    bxh_f = wx_ref[:, 1:2]
    wxh_b = wx_ref[:, 2:3]
    bxh_b = wx_ref[:, 3:4]
    wof = wx_ref[:, 4:5]
    wob = wx_ref[:, 5:6]
    half = jnp.bfloat16(0.5)

    hx_ref[:, :] = jnp.zeros_like(hx_ref)
    ones_pad = jnp.concatenate(
        [jnp.ones((1, RT), jnp.float32), jnp.zeros((1, RT), jnp.float32)],
        axis=0)
    hx_ref[pl.ds(2 * H + 2, 2), :] = ones_pad.astype(jnp.bfloat16)

    def step(t, carry):
        tb = S - 1 - t
        xf = xs_ref[pl.ds(t, 1), :].astype(jnp.bfloat16)
        xb = xs_ref[pl.ds(tb, 1), :].astype(jnp.bfloat16)
        hx_ref[pl.ds(2 * H, 2), :] = jnp.concatenate([xf, xb], axis=0)
        gf32 = jnp.dot(w, hx_ref[:, :],
                       preferred_element_type=jnp.float32)
        g = gf32[0:6 * H, :].astype(jnp.bfloat16)

        tpf = jnp.maximum(t - 1, 0)
        tpb = jnp.minimum(S - t, S - 1)
        pf_ref[pl.ds(tpf, 1), :] = gf32[6 * H:6 * H + 1, :]
        pb_ref[pl.ds(tpb, 1), :] = gf32[6 * H + 1:6 * H + 2, :]

        th_f = jnp.tanh(g[0:2 * H, :])
        hhp_f = g[2 * H:3 * H, :]
        xh_f = wxh_f * xf + bxh_f
        cf = jnp.tanh(xh_f + hhp_f + hhp_f * th_f[H:2 * H, :])
        hf = hx_ref[pl.ds(0, H), :]
        hfn = half * (hf + cf + th_f[0:H, :] * (cf - hf))

        th_b = jnp.tanh(g[3 * H:5 * H, :])
        hhp_b = g[5 * H:6 * H, :]
        xh_b = wxh_b * xb + bxh_b
        cb = jnp.tanh(xh_b + hhp_b + hhp_b * th_b[H:2 * H, :])
        hb = hx_ref[pl.ds(H, H), :]
        hbn = half * (hb + cb + th_b[0:H, :] * (cb - hb))

        hx_ref[pl.ds(0, 2 * H), :] = jnp.concatenate([hfn, hbn], axis=0)
        return carry

    jax.lax.fori_loop(0, S, step, 0)

    hfin = hx_ref[pl.ds(0, 2 * H), :]
    pf_ref[pl.ds(S - 1, 1), :] = jnp.sum(
        hfin[0:H, :] * wof, axis=0, keepdims=True).astype(jnp.float32)
    pb_ref[pl.ds(0, 1), :] = jnp.sum(
        hfin[H:2 * H, :] * wob, axis=0, keepdims=True).astype(jnp.float32)

    xs = xs_ref[:, :]
    m = ms_ref[:, :]
    imp = pf_ref[:, :] + pb_ref[:, :] + bout_ref[0, 0]
    out_ref[:, :] = m * xs + (1.0 - m) * imp


def _pack_weights(Wf, Uf, bf, Wb, Ub, bb, Wout, H):
    w = jnp.zeros((_M, _K), jnp.float32)
    UfT, UbT = Uf.T, Ub.T
    w = w.at[0:3 * H, 0:H].set(0.5 * UfT)
    w = w.at[3 * H:6 * H, H:2 * H].set(0.5 * UbT)
    w = w.at[0:2 * H, 2 * H].set(0.5 * Wf[0, 0:2 * H])
    w = w.at[3 * H:5 * H, 2 * H + 1].set(0.5 * Wb[0, 0:2 * H])
    w = w.at[0:2 * H, 2 * H + 2].set(0.5 * bf[0:2 * H])
    w = w.at[3 * H:5 * H, 2 * H + 2].set(0.5 * bb[0:2 * H])
    w = w.at[6 * H, 0:H].set(Wout[0:H, 0])
    w = w.at[6 * H + 1, H:2 * H].set(Wout[H:2 * H, 0])
    return w.astype(jnp.bfloat16)


def kernel(x, mask, Wf, Uf, bf, Wb, Ub, bb, Wout, bout):
    B, S, N, C = x.shape
    H = Uf.shape[0]
    R = B * N
    RT = 4096
    G = R // RT

    xs = x.transpose(1, 0, 2, 3).reshape(S, R)
    ms = mask.astype(jnp.float32).transpose(1, 0, 2, 3).reshape(S, R)

    w = _pack_weights(Wf, Uf, bf, Wb, Ub, bb, Wout, H)
    wx = jnp.stack([Wf[0, 2 * H:3 * H], bf[2 * H:3 * H],
                    Wb[0, 2 * H:3 * H], bb[2 * H:3 * H],
                    Wout[0:H, 0], Wout[H:2 * H, 0]],
                   axis=1).astype(jnp.bfloat16)
    bout2 = bout.reshape(1, 1)

    full = lambda shape: pl.BlockSpec(shape, lambda i: (0, 0))
    tile = pl.BlockSpec((S, RT), lambda i: (0, i))

    out = pl.pallas_call(
        _bigru_kernel,
        grid=(G,),
        in_specs=[
            tile,
            tile,
            full((_M, _K)),
            full((H, 6)),
            full((1, 1)),
        ],
        out_specs=tile,
        out_shape=jax.ShapeDtypeStruct((S, R), jnp.float32),
        scratch_shapes=[
            pltpu.VMEM((_K, RT), jnp.bfloat16),
            pltpu.VMEM((S, RT), jnp.float32),
            pltpu.VMEM((S, RT), jnp.float32),
        ],
        compiler_params=pltpu.CompilerParams(
            dimension_semantics=("arbitrary",),
        ),
    )(xs, ms, w, wx, bout2)

    return out.reshape(S, B, N, C).transpose(1, 0, 2, 3)


# speedup vs baseline: 1.1932x; 1.0594x over previous
import jax
import jax.numpy as jnp
from jax.experimental import pallas as pl
from jax.experimental.pallas import tpu as pltpu

_K = 144
_M = 392


def _bigru_kernel(xs_ref, ms_ref, w_ref, wx_ref, bout_ref,
                  out_ref, hx_ref, pf_ref, pb_ref):
    S = xs_ref.shape[0]
    H = wx_ref.shape[0]
    RT = xs_ref.shape[1]

    w = w_ref[:, :]
    wxh_f = wx_ref[:, 0:1]
    bxh_f = wx_ref[:, 1:2]
    wxh_b = wx_ref[:, 2:3]
    bxh_b = wx_ref[:, 3:4]
    wof = wx_ref[:, 4:5]
    wob = wx_ref[:, 5:6]
    half = jnp.bfloat16(0.5)

    hx_ref[:, :] = jnp.zeros_like(hx_ref)
    ones_pad = jnp.concatenate(
        [jnp.ones((1, RT), jnp.float32), jnp.zeros((1, RT), jnp.float32)],
        axis=0)
    hx_ref[pl.ds(2 * H + 2, 2), :] = ones_pad.astype(jnp.bfloat16)

    def step(t, carry):
        tb = S - 1 - t
        xf = xs_ref[pl.ds(t, 1), :].astype(jnp.bfloat16)
        xb = xs_ref[pl.ds(tb, 1), :].astype(jnp.bfloat16)
        hx_ref[pl.ds(2 * H, 2), :] = jnp.concatenate([xf, xb], axis=0)
        gf32 = jnp.dot(w, hx_ref[:, :],
                       preferred_element_type=jnp.float32)
        g = gf32[0:6 * H, :].astype(jnp.bfloat16)

        tpf = jnp.maximum(t - 1, 0)
        tpb = jnp.minimum(S - t, S - 1)
        pf_ref[pl.ds(tpf, 1), :] = gf32[6 * H:6 * H + 1, :]
        pb_ref[pl.ds(tpb, 1), :] = gf32[6 * H + 1:6 * H + 2, :]

        th_f = jnp.tanh(g[0:2 * H, :])
        hhp_f = g[2 * H:3 * H, :]
        xh_f = wxh_f * xf + bxh_f
        cf = jnp.tanh(xh_f + hhp_f + hhp_f * th_f[H:2 * H, :])
        hf = hx_ref[pl.ds(0, H), :]
        hfn = half * (hf + cf + th_f[0:H, :] * (cf - hf))

        th_b = jnp.tanh(g[3 * H:5 * H, :])
        hhp_b = g[5 * H:6 * H, :]
        xh_b = wxh_b * xb + bxh_b
        cb = jnp.tanh(xh_b + hhp_b + hhp_b * th_b[H:2 * H, :])
        hb = hx_ref[pl.ds(H, H), :]
        hbn = half * (hb + cb + th_b[0:H, :] * (cb - hb))

        hx_ref[pl.ds(0, 2 * H), :] = jnp.concatenate([hfn, hbn], axis=0)
        return carry

    jax.lax.fori_loop(0, S, step, 0)

    hfin = hx_ref[pl.ds(0, 2 * H), :]
    pf_ref[pl.ds(S - 1, 1), :] = jnp.sum(
        hfin[0:H, :] * wof, axis=0, keepdims=True).astype(jnp.float32)
    pb_ref[pl.ds(0, 1), :] = jnp.sum(
        hfin[H:2 * H, :] * wob, axis=0, keepdims=True).astype(jnp.float32)

    xs = xs_ref[:, :]
    m = ms_ref[:, :]
    imp = pf_ref[:, :] + pb_ref[:, :] + bout_ref[0, 0]
    out_ref[:, :] = m * xs + (1.0 - m) * imp


def _pack_weights(Wf, Uf, bf, Wb, Ub, bb, Wout, H):
    w = jnp.zeros((_M, _K), jnp.float32)
    UfT, UbT = Uf.T, Ub.T
    w = w.at[0:3 * H, 0:H].set(0.5 * UfT)
    w = w.at[3 * H:6 * H, H:2 * H].set(0.5 * UbT)
    w = w.at[0:2 * H, 2 * H].set(0.5 * Wf[0, 0:2 * H])
    w = w.at[3 * H:5 * H, 2 * H + 1].set(0.5 * Wb[0, 0:2 * H])
    w = w.at[0:2 * H, 2 * H + 2].set(0.5 * bf[0:2 * H])
    w = w.at[3 * H:5 * H, 2 * H + 2].set(0.5 * bb[0:2 * H])
    w = w.at[6 * H, 0:H].set(Wout[0:H, 0])
    w = w.at[6 * H + 1, H:2 * H].set(Wout[H:2 * H, 0])
    return w.astype(jnp.bfloat16)


def kernel(x, mask, Wf, Uf, bf, Wb, Ub, bb, Wout, bout):
    B, S, N, C = x.shape
    H = Uf.shape[0]
    R = B * N
    RT = 8192
    G = R // RT

    xs = x.transpose(1, 0, 2, 3).reshape(S, R)
    ms = mask.astype(jnp.float32).transpose(1, 0, 2, 3).reshape(S, R)

    w = _pack_weights(Wf, Uf, bf, Wb, Ub, bb, Wout, H)
    wx = jnp.stack([Wf[0, 2 * H:3 * H], bf[2 * H:3 * H],
                    Wb[0, 2 * H:3 * H], bb[2 * H:3 * H],
                    Wout[0:H, 0], Wout[H:2 * H, 0]],
                   axis=1).astype(jnp.bfloat16)
    bout2 = bout.reshape(1, 1)

    full = lambda shape: pl.BlockSpec(shape, lambda i: (0, 0))
    tile = pl.BlockSpec((S, RT), lambda i: (0, i))

    out = pl.pallas_call(
        _bigru_kernel,
        grid=(G,),
        in_specs=[
            tile,
            tile,
            full((_M, _K)),
            full((H, 6)),
            full((1, 1)),
        ],
        out_specs=tile,
        out_shape=jax.ShapeDtypeStruct((S, R), jnp.float32),
        scratch_shapes=[
            pltpu.VMEM((_K, RT), jnp.bfloat16),
            pltpu.VMEM((S, RT), jnp.float32),
            pltpu.VMEM((S, RT), jnp.float32),
        ],
        compiler_params=pltpu.CompilerParams(
            dimension_semantics=("arbitrary",),
        ),
    )(xs, ms, w, wx, bout2)

    return out.reshape(S, B, N, C).transpose(1, 0, 2, 3)
